# Initial kernel scaffold; baseline (speedup 1.0000x reference)
import jax
import jax.numpy as jnp
from jax.experimental import pallas as pl


def kernel(gene, miRNA, rna_w, rna_b, mi_w, mi_b, hw_r_nl_w, hw_r_nl_b, hw_r_l_w, hw_r_l_b, hw_r_g_w, hw_r_g_b, hw_m_nl_w, hw_m_nl_b, hw_m_l_w, hw_m_l_b, hw_m_g_w, hw_m_g_b, rna_lin_w, mi_lin_w, bn_g, bn_b, fuse_w, fuse_b, c1_w, c1_b, c2_w, c2_b):
    raise NotImplementedError("write your pallas kernel here")


# speedup vs baseline: 7.5448x; 7.5448x over previous
import functools

import jax
import jax.numpy as jnp
import numpy as np
from jax import lax
from jax.experimental import pallas as pl
from jax.experimental.pallas import tpu as pltpu
from jax.experimental.pallas import tpu_sc as plsc

B = 4096
NL = 5
RB = 256
NB = B // RB
KS = (5, 3, 3)
S0 = (0, 5, 8)
NSLOT = 11
NSLOTP = 12
EPS = float(np.finfo(np.float64).eps)
XP = 48
XG = 128

NC, NSUB, LN = 2, 16, 16
NW = NC * NSUB
RPW = B // NW
NCH = RPW // LN


def _dot_t(a, w):
    return lax.dot_general(a, w, (((1,), (1,)), ((), ())),
                           preferred_element_type=jnp.float32)


def _l2n(x):
    n = jnp.sqrt(jnp.sum(x * x, axis=1, keepdims=True))
    return x / jnp.maximum(n, 1e-12)


def _tc1_body(gene_ref, mi_ref, rna_w_ref, rna_b_ref, mi_w_ref, mi_b_ref,
              rnl_w, rnl_b, rl_w, rl_b, rg_w, rg_b,
              mnl_w, mnl_b, ml_w, ml_b, mg_w, mg_b,
              rna_lin_ref, mi_lin_ref, bn_g_ref, bn_b_ref,
              fuse_w_ref, fuse_b_ref,
              rn_ref, mn_ref, fn_ref, xp_ref):
    def highway(x, nl_w, nl_b, l_w, l_b, g_w, g_b):
        for i in range(NL):
            gate = jax.nn.sigmoid(_dot_t(x, g_w[i]) + g_b[i][None, :])
            nonlinear = jax.nn.relu(_dot_t(x, nl_w[i]) + nl_b[i][None, :])
            linear = jax.nn.relu(_dot_t(x, l_w[i]) + l_b[i][None, :])
            x = gate * nonlinear + (1.0 - gate) * linear
        return x

    rna_f = highway(
        jnp.tanh(_dot_t(gene_ref[...], rna_w_ref[...]) + rna_b_ref[...][None, :]),
        rnl_w[...], rnl_b[...], rl_w[...], rl_b[...], rg_w[...], rg_b[...])
    mi_f = highway(
        jnp.tanh(_dot_t(mi_ref[...], mi_w_ref[...]) + mi_b_ref[...][None, :]),
        mnl_w[...], mnl_b[...], ml_w[...], ml_b[...], mg_w[...], mg_b[...])

    fbn = _dot_t(rna_f, rna_lin_ref[...]) * _dot_t(mi_f, mi_lin_ref[...])
    fused = jnp.concatenate([rna_f + mi_f, fbn], axis=1)

    rn_ref[...] = _l2n(rna_f)
    mn_ref[...] = _l2n(mi_f)
    fn_ref[...] = _l2n(fused)

    bn = (fused / jnp.sqrt(jnp.float32(1.0 + 1e-5))) * bn_g_ref[...][None, :] \
        + bn_b_ref[...][None, :]
    x = jnp.tanh(_dot_t(bn, fuse_w_ref[...]) + fuse_b_ref[...][None, :])
    xp_ref[...] = jnp.pad(x, ((0, 0), (0, XG - 40)))


def _tc1(gene, miRNA, rna_w, rna_b, mi_w, mi_b,
         rnl_w, rnl_b, rl_w, rl_b, rg_w, rg_b,
         mnl_w, mnl_b, ml_w, ml_b, mg_w, mg_b,
         rna_lin_w, mi_lin_w, bn_g, bn_b, fuse_w, fuse_b):
    def full(a):
        return pl.BlockSpec(a.shape, lambda i: (0,) * a.ndim)
    in_specs = [
        pl.BlockSpec((RB, 6000), lambda i: (i, 0)),
        pl.BlockSpec((RB, 600), lambda i: (i, 0)),
        full(rna_w), full(rna_b), full(mi_w), full(mi_b),
        full(rnl_w), full(rnl_b), full(rl_w), full(rl_b), full(rg_w), full(rg_b),
        full(mnl_w), full(mnl_b), full(ml_w), full(ml_b), full(mg_w), full(mg_b),
        full(rna_lin_w), full(mi_lin_w), full(bn_g), full(bn_b),
        full(fuse_w), full(fuse_b),
    ]
    out_specs = [
        pl.BlockSpec((RB, 40), lambda i: (i, 0)),
        pl.BlockSpec((RB, 40), lambda i: (i, 0)),
        pl.BlockSpec((RB, 60), lambda i: (i, 0)),
        pl.BlockSpec((RB, XG), lambda i: (i, 0)),
    ]
    out_shape = [
        jax.ShapeDtypeStruct((B, 40), jnp.float32),
        jax.ShapeDtypeStruct((B, 40), jnp.float32),
        jax.ShapeDtypeStruct((B, 60), jnp.float32),
        jax.ShapeDtypeStruct((B, XG), jnp.float32),
    ]
    return pl.pallas_call(
        _tc1_body, grid=(NB,), in_specs=in_specs, out_specs=out_specs,
        out_shape=out_shape)(
            gene, miRNA, rna_w, rna_b, mi_w, mi_b,
            rnl_w, rnl_b, rl_w, rl_b, rg_w, rg_b,
            mnl_w, mnl_b, ml_w, ml_b, mg_w, mg_b,
            rna_lin_w, mi_lin_w, bn_g, bn_b, fuse_w, fuse_b)


def _tc2_body(rnb_ref, rn_ref, mnb_ref, mn_ref, fnb_ref, fn_ref,
              idxt_ref, at_ref, wt_ref, u_ref):
    pid = pl.program_id(0)
    iota = lax.broadcasted_iota(jnp.int32, (RB, B), 1)
    for g, (blk_ref, ful_ref) in enumerate(
            [(rnb_ref, rn_ref), (mnb_ref, mn_ref), (fnb_ref, fn_ref)]):
        fb = blk_ref[...]
        ff = ful_ref[...]
        sqf = jnp.sum(ff * ff, axis=1)
        sqb = jnp.sum(fb * fb, axis=1)
        g2 = 2.0 * _dot_t(fb, ff)
        key = g2 - sqf[None, :]
        k = KS[g]
        for t in range(k):
            m = jnp.max(key, axis=1)
            eq = key == m[:, None]
            idxv = jnp.min(jnp.where(eq, iota, B), axis=1)
            sel = iota == idxv[:, None]
            g2sel = jnp.sum(jnp.where(sel, g2, 0.0), axis=1)
            key = jnp.where(sel, -jnp.inf, key)
            idxt_ref[S0[g] + t, :] = idxv
            at_ref[S0[g] + t, :] = g2sel - sqb
            wt_ref[S0[g] + t, :] = jnp.exp(-(sqb - m) / 20.0)
            if t == k - 1:
                u_ref[g, :] = m
    idxt_ref[NSLOT, :] = pid * RB + lax.iota(jnp.int32, RB)


def _tc2(rn, mn, fn):
    def blk(d):
        return pl.BlockSpec((RB, d), lambda i: (i, 0))

    def ful(d):
        return pl.BlockSpec((B, d), lambda i: (0, 0))
    return pl.pallas_call(
        _tc2_body, grid=(NB,),
        in_specs=[blk(40), ful(40), blk(40), ful(40), blk(60), ful(60)],
        out_specs=[
            pl.BlockSpec((NSLOTP, RB), lambda i: (0, i)),
            pl.BlockSpec((NSLOT, RB), lambda i: (0, i)),
            pl.BlockSpec((NSLOT, RB), lambda i: (0, i)),
            pl.BlockSpec((3, RB), lambda i: (0, i)),
        ],
        out_shape=[
            jax.ShapeDtypeStruct((NSLOTP, B), jnp.int32),
            jax.ShapeDtypeStruct((NSLOT, B), jnp.float32),
            jax.ShapeDtypeStruct((NSLOT, B), jnp.float32),
            jax.ShapeDtypeStruct((3, B), jnp.float32),
        ])(rn, rn, mn, mn, fn, fn)


_SC_MESH = plsc.VectorSubcoreMesh(core_axis_name="c", subcore_axis_name="s")


def _wid_base():
    return (lax.axis_index("s") * NC + lax.axis_index("c")) * RPW


def _sc1_body(idxt_hbm, at_hbm, wt_hbm, u_hbm,
              mvt_hbm, c_hbm,
              u_v, idx_v, a_v, w_v, mv_v, c_v):
    base = _wid_base()
    pltpu.sync_copy(u_hbm, u_v)
    pltpu.sync_copy(idxt_hbm.at[:, pl.ds(base, RPW)], idx_v)
    pltpu.sync_copy(at_hbm.at[:, pl.ds(base, RPW)], a_v)
    pltpu.sync_copy(wt_hbm.at[:, pl.ds(base, RPW)], w_v)
    for g in range(3):
        goff = jnp.full((LN,), g * B, jnp.int32)
        for ch in range(NCH):
            sl = pl.ds(ch * LN, LN)
            csum = jnp.zeros((LN,), jnp.float32)
            for s in range(S0[g], S0[g] + KS[g]):
                j = idx_v[s, sl]
                uj = plsc.load_gather(u_v, [j + goff])
                keep = a_v[s, sl] >= uj
                mv = jnp.where(keep, w_v[s, sl], 0.0)
                mv_v[s, sl] = mv
                csum = csum + mv
            c_v[g, sl] = csum
    pltpu.sync_copy(mv_v, mvt_hbm.at[:, pl.ds(base, RPW)])
    pltpu.sync_copy(c_v, c_hbm.at[:, pl.ds(base, RPW)])


def _sc1(idxt, at, wt, u_flat):
    f = functools.partial(
        pl.kernel, mesh=_SC_MESH,
        compiler_params=pltpu.CompilerParams(needs_layout_passes=False),
        out_type=[
            jax.ShapeDtypeStruct((NSLOT, B), jnp.float32),
            jax.ShapeDtypeStruct((3, B), jnp.float32),
        ],
        scratch_types=[
            pltpu.VMEM((3 * B,), jnp.float32),
            pltpu.VMEM((NSLOTP, RPW), jnp.int32),
            pltpu.VMEM((NSLOT, RPW), jnp.float32),
            pltpu.VMEM((NSLOT, RPW), jnp.float32),
            pltpu.VMEM((NSLOT, RPW), jnp.float32),
            pltpu.VMEM((3, RPW), jnp.float32),
        ])
    return f(_sc1_body)(idxt, at, wt, u_flat)


def _sc2_body(idxt_hbm, mvt_hbm, c_hbm,
              wnt_hbm, ds2_hbm,
              c_v, idx_v, mv_v, wn_v, ds2_v):
    base = _wid_base()
    pltpu.sync_copy(c_hbm, c_v)
    pltpu.sync_copy(idxt_hbm.at[:, pl.ds(base, RPW)], idx_v)
    pltpu.sync_copy(mvt_hbm.at[:, pl.ds(base, RPW)], mv_v)
    for ch in range(NCH):
        sl = pl.ds(ch * LN, LN)
        dsum = jnp.full((LN,), 1.0, jnp.float32)
        for g in range(3):
            goff = jnp.full((LN,), g * B, jnp.int32)
            for s in range(S0[g], S0[g] + KS[g]):
                j = idx_v[s, sl]
                cj = plsc.load_gather(c_v, [j + goff])
                wn = mv_v[s, sl] / cj
                wn_v[s, sl] = wn
                dsum = dsum + wn
        wn_v[NSLOT, sl] = jnp.full((LN,), 1.0, jnp.float32)
        ds2_v[sl] = 1.0 / (dsum + EPS)
    pltpu.sync_copy(wn_v, wnt_hbm.at[:, pl.ds(base, RPW)])
    pltpu.sync_copy(ds2_v, ds2_hbm.at[pl.ds(base, RPW)])


def _sc2(idxt, mvt, c_flat):
    f = functools.partial(
        pl.kernel, mesh=_SC_MESH,
        compiler_params=pltpu.CompilerParams(needs_layout_passes=False),
        out_type=[
            jax.ShapeDtypeStruct((NSLOTP, B), jnp.float32),
            jax.ShapeDtypeStruct((B,), jnp.float32),
        ],
        scratch_types=[
            pltpu.VMEM((3 * B,), jnp.float32),
            pltpu.VMEM((NSLOTP, RPW), jnp.int32),
            pltpu.VMEM((NSLOT, RPW), jnp.float32),
            pltpu.VMEM((NSLOTP, RPW), jnp.float32),
            pltpu.VMEM((RPW,), jnp.float32),
        ])
    return f(_sc2_body)(idxt, mvt, c_flat)


def _sc3_body(idxt_hbm, wnt_hbm, ds2_hbm, xp_hbm,
              h_hbm,
              ds2_v, idx_v, wn_v, xg_v, h_v):
    base = _wid_base()
    pltpu.sync_copy(ds2_hbm, ds2_v)
    pltpu.sync_copy(idxt_hbm.at[:, pl.ds(base, RPW)], idx_v)
    pltpu.sync_copy(wnt_hbm.at[:, pl.ds(base, RPW)], wn_v)
    for r in range(RPW):
        for q in range(XP // LN):
            h_v[r, pl.ds(q * LN, LN)] = jnp.zeros((LN,), jnp.float32)

    @pl.loop(0, NSLOTP)
    def _slot(s):
        pltpu.sync_copy(xp_hbm.at[idx_v.at[s]], xg_v)
        for ch in range(NCH):
            sl = pl.ds(ch * LN, LN)
            j = idx_v[s, sl]
            dj = plsc.load_gather(ds2_v, [j])
            coef = wn_v[s, sl] * dj
            for t in range(LN):
                cf = coef[t]
                r = ch * LN + t
                for q in range(XP // LN):
                    qs = pl.ds(q * LN, LN)
                    h_v[r, qs] = h_v[r, qs] + cf * xg_v[r, qs]
    pltpu.sync_copy(h_v, h_hbm.at[pl.ds(base, RPW), :])


def _sc3(idxt, wnt, ds2, xp):
    f = functools.partial(
        pl.kernel, mesh=_SC_MESH,
        compiler_params=pltpu.CompilerParams(needs_layout_passes=False),
        out_type=jax.ShapeDtypeStruct((B, XP), jnp.float32),
        scratch_types=[
            pltpu.VMEM((B,), jnp.float32),
            pltpu.VMEM((NSLOTP, RPW), jnp.int32),
            pltpu.VMEM((NSLOTP, RPW), jnp.float32),
            pltpu.VMEM((RPW, XG), jnp.float32),
            pltpu.VMEM((RPW, XP), jnp.float32),
        ])
    return f(_sc3_body)(idxt, wnt, ds2, xp)


def _tc3_body(h_ref, c1_w_ref, c1_b_ref, c2_w_ref, c2_b_ref, out_ref):
    h = h_ref[...][:, :40]
    a1 = jax.nn.relu(_dot_t(h, c1_w_ref[...]) + c1_b_ref[...][None, :])
    res = jnp.sum(a1 * c2_w_ref[...], axis=1, keepdims=True)
    out_ref[...] = res + c2_b_ref[...]


def _tc3(ht, c1_w, c1_b, c2_w, c2_b):
    def full(a):
        return pl.BlockSpec(a.shape, lambda: (0,) * a.ndim)
    return pl.pallas_call(
        _tc3_body,
        in_specs=[full(ht), full(c1_w), full(c1_b), full(c2_w), full(c2_b)],
        out_specs=pl.BlockSpec((B, 1), lambda: (0, 0)),
        out_shape=jax.ShapeDtypeStruct((B, 1), jnp.float32),
    )(ht, c1_w, c1_b, c2_w, c2_b)


def kernel(gene, miRNA, rna_w, rna_b, mi_w, mi_b,
           hw_r_nl_w, hw_r_nl_b, hw_r_l_w, hw_r_l_b, hw_r_g_w, hw_r_g_b,
           hw_m_nl_w, hw_m_nl_b, hw_m_l_w, hw_m_l_b, hw_m_g_w, hw_m_g_b,
           rna_lin_w, mi_lin_w, bn_g, bn_b, fuse_w, fuse_b,
           c1_w, c1_b, c2_w, c2_b):
    gene = gene.reshape(gene.shape[0], -1)
    miRNA = miRNA.reshape(miRNA.shape[0], -1)
    rn, mn, fn, xp = _tc1(gene, miRNA, rna_w, rna_b, mi_w, mi_b,
                          hw_r_nl_w, hw_r_nl_b, hw_r_l_w, hw_r_l_b,
                          hw_r_g_w, hw_r_g_b,
                          hw_m_nl_w, hw_m_nl_b, hw_m_l_w, hw_m_l_b,
                          hw_m_g_w, hw_m_g_b,
                          rna_lin_w, mi_lin_w, bn_g, bn_b, fuse_w, fuse_b)
    idxt, at, wt, u = _tc2(rn, mn, fn)
    mvt, c = _sc1(idxt, at, wt, u.reshape(3 * B))
    wnt, ds2 = _sc2(idxt, mvt, c.reshape(3 * B))
    h = _sc3(idxt, wnt, ds2, xp)
    return _tc3(h, c1_w, c1_b, c2_w, c2_b)


# speedup vs baseline: 8.6607x; 1.1479x over previous
import functools

import jax
import jax.numpy as jnp
import numpy as np
from jax import lax
from jax.experimental import pallas as pl
from jax.experimental.pallas import tpu as pltpu
from jax.experimental.pallas import tpu_sc as plsc

B = 4096
NL = 5
RB = 256
NB = B // RB
KS = (5, 3, 3)
S0 = (0, 5, 8)
NSLOT = 11
NSLOTP = 12
EPS = float(np.finfo(np.float64).eps)
XP = 48
XG = 128

NC, NSUB, LN = 2, 16, 16
NW = NC * NSUB
RPW = B // NW
NCH = RPW // LN


def _dot_t(a, w):
    return lax.dot_general(a, w, (((1,), (1,)), ((), ())),
                           preferred_element_type=jnp.float32)


def _l2n(x):
    n = jnp.sqrt(jnp.sum(x * x, axis=1, keepdims=True))
    return x / jnp.maximum(n, 1e-12)


def _tc1_body(gene_ref, mi_ref, rna_w_ref, rna_b_ref, mi_w_ref, mi_b_ref,
              rnl_w, rnl_b, rl_w, rl_b, rg_w, rg_b,
              mnl_w, mnl_b, ml_w, ml_b, mg_w, mg_b,
              rna_lin_ref, mi_lin_ref, bn_g_ref, bn_b_ref,
              fuse_w_ref, fuse_b_ref,
              rn_ref, mn_ref, fn_ref, xp_ref):
    def highway(x, nl_w, nl_b, l_w, l_b, g_w, g_b):
        for i in range(NL):
            gate = jax.nn.sigmoid(_dot_t(x, g_w[i]) + g_b[i][None, :])
            nonlinear = jax.nn.relu(_dot_t(x, nl_w[i]) + nl_b[i][None, :])
            linear = jax.nn.relu(_dot_t(x, l_w[i]) + l_b[i][None, :])
            x = gate * nonlinear + (1.0 - gate) * linear
        return x

    rna_f = highway(
        jnp.tanh(_dot_t(gene_ref[...], rna_w_ref[...]) + rna_b_ref[...][None, :]),
        rnl_w[...], rnl_b[...], rl_w[...], rl_b[...], rg_w[...], rg_b[...])
    mi_f = highway(
        jnp.tanh(_dot_t(mi_ref[...], mi_w_ref[...]) + mi_b_ref[...][None, :]),
        mnl_w[...], mnl_b[...], ml_w[...], ml_b[...], mg_w[...], mg_b[...])

    fbn = _dot_t(rna_f, rna_lin_ref[...]) * _dot_t(mi_f, mi_lin_ref[...])
    fused = jnp.concatenate([rna_f + mi_f, fbn], axis=1)

    rn_ref[...] = _l2n(rna_f)
    mn_ref[...] = _l2n(mi_f)
    fn_ref[...] = _l2n(fused)

    bn = (fused / jnp.sqrt(jnp.float32(1.0 + 1e-5))) * bn_g_ref[...][None, :] \
        + bn_b_ref[...][None, :]
    x = jnp.tanh(_dot_t(bn, fuse_w_ref[...]) + fuse_b_ref[...][None, :])
    xp_ref[...] = jnp.pad(x, ((0, 0), (0, XG - 40)))


def _tc1(gene, miRNA, rna_w, rna_b, mi_w, mi_b,
         rnl_w, rnl_b, rl_w, rl_b, rg_w, rg_b,
         mnl_w, mnl_b, ml_w, ml_b, mg_w, mg_b,
         rna_lin_w, mi_lin_w, bn_g, bn_b, fuse_w, fuse_b):
    def full(a):
        return pl.BlockSpec(a.shape, lambda i: (0,) * a.ndim)
    in_specs = [
        pl.BlockSpec((RB, 6000), lambda i: (i, 0)),
        pl.BlockSpec((RB, 600), lambda i: (i, 0)),
        full(rna_w), full(rna_b), full(mi_w), full(mi_b),
        full(rnl_w), full(rnl_b), full(rl_w), full(rl_b), full(rg_w), full(rg_b),
        full(mnl_w), full(mnl_b), full(ml_w), full(ml_b), full(mg_w), full(mg_b),
        full(rna_lin_w), full(mi_lin_w), full(bn_g), full(bn_b),
        full(fuse_w), full(fuse_b),
    ]
    out_specs = [
        pl.BlockSpec((RB, 40), lambda i: (i, 0)),
        pl.BlockSpec((RB, 40), lambda i: (i, 0)),
        pl.BlockSpec((RB, 60), lambda i: (i, 0)),
        pl.BlockSpec((RB, XG), lambda i: (i, 0)),
    ]
    out_shape = [
        jax.ShapeDtypeStruct((B, 40), jnp.float32),
        jax.ShapeDtypeStruct((B, 40), jnp.float32),
        jax.ShapeDtypeStruct((B, 60), jnp.float32),
        jax.ShapeDtypeStruct((B, XG), jnp.float32),
    ]
    return pl.pallas_call(
        _tc1_body, grid=(NB,), in_specs=in_specs, out_specs=out_specs,
        out_shape=out_shape)(
            gene, miRNA, rna_w, rna_b, mi_w, mi_b,
            rnl_w, rnl_b, rl_w, rl_b, rg_w, rg_b,
            mnl_w, mnl_b, ml_w, ml_b, mg_w, mg_b,
            rna_lin_w, mi_lin_w, bn_g, bn_b, fuse_w, fuse_b)


def _tc2_body(rnb_ref, rn_ref, mnb_ref, mn_ref, fnb_ref, fn_ref,
              idxt_ref, at_ref, wt_ref, u_ref):
    pid = pl.program_id(0)
    iota = lax.broadcasted_iota(jnp.int32, (RB, B), 1)
    for g, (blk_ref, ful_ref) in enumerate(
            [(rnb_ref, rn_ref), (mnb_ref, mn_ref), (fnb_ref, fn_ref)]):
        fb = blk_ref[...]
        ff = ful_ref[...]
        sqf = jnp.sum(ff * ff, axis=1)
        sqb = jnp.sum(fb * fb, axis=1)
        d2 = (sqb[:, None] + sqf[None, :]) - 2.0 * _dot_t(fb, ff)
        k = KS[g]
        for t in range(k):
            m = jnp.min(d2, axis=1)
            eq = d2 == m[:, None]
            idxv = jnp.min(jnp.where(eq, iota, B), axis=1)
            d2 = jnp.where(eq, jnp.inf, d2)
            idxt_ref[S0[g] + t, :] = idxv
            at_ref[S0[g] + t, :] = m
            wt_ref[S0[g] + t, :] = jnp.exp(-m / 20.0)
            if t == k - 1:
                u_ref[g, :] = m
    idxt_ref[NSLOT, :] = pid * RB + lax.iota(jnp.int32, RB)


def _tc2(rn, mn, fn):
    def blk(d):
        return pl.BlockSpec((RB, d), lambda i: (i, 0))

    def ful(d):
        return pl.BlockSpec((B, d), lambda i: (0, 0))
    return pl.pallas_call(
        _tc2_body, grid=(NB,),
        in_specs=[blk(40), ful(40), blk(40), ful(40), blk(60), ful(60)],
        out_specs=[
            pl.BlockSpec((NSLOTP, RB), lambda i: (0, i)),
            pl.BlockSpec((NSLOT, RB), lambda i: (0, i)),
            pl.BlockSpec((NSLOT, RB), lambda i: (0, i)),
            pl.BlockSpec((3, RB), lambda i: (0, i)),
        ],
        out_shape=[
            jax.ShapeDtypeStruct((NSLOTP, B), jnp.int32),
            jax.ShapeDtypeStruct((NSLOT, B), jnp.float32),
            jax.ShapeDtypeStruct((NSLOT, B), jnp.float32),
            jax.ShapeDtypeStruct((3, B), jnp.float32),
        ])(rn, rn, mn, mn, fn, fn)


_SC_MESH = plsc.VectorSubcoreMesh(core_axis_name="c", subcore_axis_name="s")


def _wid_base():
    return (lax.axis_index("s") * NC + lax.axis_index("c")) * RPW


def _sc1_body(idxt_hbm, at_hbm, wt_hbm, u_hbm,
              mvt_hbm, c_hbm,
              u_v, idx_v, a_v, w_v, mv_v, c_v):
    base = _wid_base()
    pltpu.sync_copy(u_hbm, u_v)
    pltpu.sync_copy(idxt_hbm.at[:, pl.ds(base, RPW)], idx_v)
    pltpu.sync_copy(at_hbm.at[:, pl.ds(base, RPW)], a_v)
    pltpu.sync_copy(wt_hbm.at[:, pl.ds(base, RPW)], w_v)
    for g in range(3):
        goff = jnp.full((LN,), g * B, jnp.int32)
        for ch in range(NCH):
            sl = pl.ds(ch * LN, LN)
            csum = jnp.zeros((LN,), jnp.float32)
            for s in range(S0[g], S0[g] + KS[g]):
                j = idx_v[s, sl]
                uj = plsc.load_gather(u_v, [j + goff])
                keep = a_v[s, sl] <= uj
                mv = jnp.where(keep, w_v[s, sl], 0.0)
                mv_v[s, sl] = mv
                csum = csum + mv
            c_v[g, sl] = csum
    pltpu.sync_copy(mv_v, mvt_hbm.at[:, pl.ds(base, RPW)])
    pltpu.sync_copy(c_v, c_hbm.at[:, pl.ds(base, RPW)])


def _sc1(idxt, at, wt, u_flat):
    f = functools.partial(
        pl.kernel, mesh=_SC_MESH,
        compiler_params=pltpu.CompilerParams(needs_layout_passes=False),
        out_type=[
            jax.ShapeDtypeStruct((NSLOT, B), jnp.float32),
            jax.ShapeDtypeStruct((3, B), jnp.float32),
        ],
        scratch_types=[
            pltpu.VMEM((3 * B,), jnp.float32),
            pltpu.VMEM((NSLOTP, RPW), jnp.int32),
            pltpu.VMEM((NSLOT, RPW), jnp.float32),
            pltpu.VMEM((NSLOT, RPW), jnp.float32),
            pltpu.VMEM((NSLOT, RPW), jnp.float32),
            pltpu.VMEM((3, RPW), jnp.float32),
        ])
    return f(_sc1_body)(idxt, at, wt, u_flat)


def _sc2_body(idxt_hbm, mvt_hbm, c_hbm,
              wnt_hbm, ds2_hbm,
              c_v, idx_v, mv_v, wn_v, ds2_v):
    base = _wid_base()
    pltpu.sync_copy(c_hbm, c_v)
    pltpu.sync_copy(idxt_hbm.at[:, pl.ds(base, RPW)], idx_v)
    pltpu.sync_copy(mvt_hbm.at[:, pl.ds(base, RPW)], mv_v)
    for ch in range(NCH):
        sl = pl.ds(ch * LN, LN)
        dsum = jnp.full((LN,), 1.0, jnp.float32)
        for g in range(3):
            goff = jnp.full((LN,), g * B, jnp.int32)
            for s in range(S0[g], S0[g] + KS[g]):
                j = idx_v[s, sl]
                cj = plsc.load_gather(c_v, [j + goff])
                wn = mv_v[s, sl] / cj
                wn_v[s, sl] = wn
                dsum = dsum + wn
        wn_v[NSLOT, sl] = jnp.full((LN,), 1.0, jnp.float32)
        ds2_v[sl] = 1.0 / (dsum + EPS)
    pltpu.sync_copy(wn_v, wnt_hbm.at[:, pl.ds(base, RPW)])
    pltpu.sync_copy(ds2_v, ds2_hbm.at[pl.ds(base, RPW)])


def _sc2(idxt, mvt, c_flat):
    f = functools.partial(
        pl.kernel, mesh=_SC_MESH,
        compiler_params=pltpu.CompilerParams(needs_layout_passes=False),
        out_type=[
            jax.ShapeDtypeStruct((NSLOTP, B), jnp.float32),
            jax.ShapeDtypeStruct((B,), jnp.float32),
        ],
        scratch_types=[
            pltpu.VMEM((3 * B,), jnp.float32),
            pltpu.VMEM((NSLOTP, RPW), jnp.int32),
            pltpu.VMEM((NSLOT, RPW), jnp.float32),
            pltpu.VMEM((NSLOTP, RPW), jnp.float32),
            pltpu.VMEM((RPW,), jnp.float32),
        ])
    return f(_sc2_body)(idxt, mvt, c_flat)


def _sc3_body(idxt_hbm, wnt_hbm, ds2_hbm, xp_hbm,
              h_hbm,
              ds2_v, idx_v, wn_v, xg_v, h_v):
    base = _wid_base()
    pltpu.sync_copy(ds2_hbm, ds2_v)
    pltpu.sync_copy(idxt_hbm.at[:, pl.ds(base, RPW)], idx_v)
    pltpu.sync_copy(wnt_hbm.at[:, pl.ds(base, RPW)], wn_v)
    for r in range(RPW):
        for q in range(XP // LN):
            h_v[r, pl.ds(q * LN, LN)] = jnp.zeros((LN,), jnp.float32)

    @pl.loop(0, NSLOTP)
    def _slot(s):
        pltpu.sync_copy(xp_hbm.at[idx_v.at[s]], xg_v)
        for ch in range(NCH):
            sl = pl.ds(ch * LN, LN)
            j = idx_v[s, sl]
            dj = plsc.load_gather(ds2_v, [j])
            coef = wn_v[s, sl] * dj
            for t in range(LN):
                cf = coef[t]
                r = ch * LN + t
                for q in range(XP // LN):
                    qs = pl.ds(q * LN, LN)
                    h_v[r, qs] = h_v[r, qs] + cf * xg_v[r, qs]
    pltpu.sync_copy(h_v, h_hbm.at[pl.ds(base, RPW), :])


def _sc3(idxt, wnt, ds2, xp):
    f = functools.partial(
        pl.kernel, mesh=_SC_MESH,
        compiler_params=pltpu.CompilerParams(needs_layout_passes=False),
        out_type=jax.ShapeDtypeStruct((B, XP), jnp.float32),
        scratch_types=[
            pltpu.VMEM((B,), jnp.float32),
            pltpu.VMEM((NSLOTP, RPW), jnp.int32),
            pltpu.VMEM((NSLOTP, RPW), jnp.float32),
            pltpu.VMEM((RPW, XG), jnp.float32),
            pltpu.VMEM((RPW, XP), jnp.float32),
        ])
    return f(_sc3_body)(idxt, wnt, ds2, xp)


def _tc3_body(h_ref, c1_w_ref, c1_b_ref, c2_w_ref, c2_b_ref, out_ref):
    h = h_ref[...][:, :40]
    a1 = jax.nn.relu(_dot_t(h, c1_w_ref[...]) + c1_b_ref[...][None, :])
    res = jnp.sum(a1 * c2_w_ref[...], axis=1, keepdims=True)
    out_ref[...] = res + c2_b_ref[...]


def _tc3(ht, c1_w, c1_b, c2_w, c2_b):
    def full(a):
        return pl.BlockSpec(a.shape, lambda: (0,) * a.ndim)
    return pl.pallas_call(
        _tc3_body,
        in_specs=[full(ht), full(c1_w), full(c1_b), full(c2_w), full(c2_b)],
        out_specs=pl.BlockSpec((B, 1), lambda: (0, 0)),
        out_shape=jax.ShapeDtypeStruct((B, 1), jnp.float32),
    )(ht, c1_w, c1_b, c2_w, c2_b)


def kernel(gene, miRNA, rna_w, rna_b, mi_w, mi_b,
           hw_r_nl_w, hw_r_nl_b, hw_r_l_w, hw_r_l_b, hw_r_g_w, hw_r_g_b,
           hw_m_nl_w, hw_m_nl_b, hw_m_l_w, hw_m_l_b, hw_m_g_w, hw_m_g_b,
           rna_lin_w, mi_lin_w, bn_g, bn_b, fuse_w, fuse_b,
           c1_w, c1_b, c2_w, c2_b):
    gene = gene.reshape(gene.shape[0], -1)
    miRNA = miRNA.reshape(miRNA.shape[0], -1)
    rn, mn, fn, xp = _tc1(gene, miRNA, rna_w, rna_b, mi_w, mi_b,
                          hw_r_nl_w, hw_r_nl_b, hw_r_l_w, hw_r_l_b,
                          hw_r_g_w, hw_r_g_b,
                          hw_m_nl_w, hw_m_nl_b, hw_m_l_w, hw_m_l_b,
                          hw_m_g_w, hw_m_g_b,
                          rna_lin_w, mi_lin_w, bn_g, bn_b, fuse_w, fuse_b)
    idxt, at, wt, u = _tc2(rn, mn, fn)
    mvt, c = _sc1(idxt, at, wt, u.reshape(3 * B))
    wnt, ds2 = _sc2(idxt, mvt, c.reshape(3 * B))
    h = _sc3(idxt, wnt, ds2, xp)
    return _tc3(h, c1_w, c1_b, c2_w, c2_b)


# speedup vs baseline: 8.8837x; 1.0257x over previous
import functools

import jax
import jax.numpy as jnp
import numpy as np
from jax import lax
from jax.experimental import pallas as pl
from jax.experimental.pallas import tpu as pltpu
from jax.experimental.pallas import tpu_sc as plsc

B = 4096
NL = 5
RB = 256
NB = B // RB
RB1 = 512
NB1 = B // RB1
KS = (5, 3, 3)
S0 = (0, 5, 8)
NSLOT = 11
NSLOTP = 12
EPS = float(np.finfo(np.float64).eps)
XP = 48
XG = 128

NC, NSUB, LN = 2, 16, 16
NW = NC * NSUB
RPW = B // NW
NCH = RPW // LN


def _dot_t(a, w):
    return lax.dot_general(a, w, (((1,), (1,)), ((), ())),
                           preferred_element_type=jnp.float32)


def _l2n(x):
    n = jnp.sqrt(jnp.sum(x * x, axis=1, keepdims=True))
    return x / jnp.maximum(n, 1e-12)


def _tc1_body(gene_ref, mi_ref, rna_w_ref, rna_b_ref, mi_w_ref, mi_b_ref,
              rnl_w, rnl_b, rl_w, rl_b, rg_w, rg_b,
              mnl_w, mnl_b, ml_w, ml_b, mg_w, mg_b,
              rna_lin_ref, mi_lin_ref, bn_g_ref, bn_b_ref,
              fuse_w_ref, fuse_b_ref,
              rn_ref, mn_ref, fn_ref, xp_ref):
    def highway(x, nl_w, nl_b, l_w, l_b, g_w, g_b):
        for i in range(NL):
            gate = jax.nn.sigmoid(_dot_t(x, g_w[i]) + g_b[i][None, :])
            nonlinear = jax.nn.relu(_dot_t(x, nl_w[i]) + nl_b[i][None, :])
            linear = jax.nn.relu(_dot_t(x, l_w[i]) + l_b[i][None, :])
            x = gate * nonlinear + (1.0 - gate) * linear
        return x

    rna_f = highway(
        jnp.tanh(_dot_t(gene_ref[...], rna_w_ref[...]) + rna_b_ref[...][None, :]),
        rnl_w[...], rnl_b[...], rl_w[...], rl_b[...], rg_w[...], rg_b[...])
    mi_f = highway(
        jnp.tanh(_dot_t(mi_ref[...], mi_w_ref[...]) + mi_b_ref[...][None, :]),
        mnl_w[...], mnl_b[...], ml_w[...], ml_b[...], mg_w[...], mg_b[...])

    fbn = _dot_t(rna_f, rna_lin_ref[...]) * _dot_t(mi_f, mi_lin_ref[...])
    fused = jnp.concatenate([rna_f + mi_f, fbn], axis=1)

    rn_ref[...] = _l2n(rna_f)
    mn_ref[...] = _l2n(mi_f)
    fn_ref[...] = _l2n(fused)

    bn = (fused / jnp.sqrt(jnp.float32(1.0 + 1e-5))) * bn_g_ref[...][None, :] \
        + bn_b_ref[...][None, :]
    x = jnp.tanh(_dot_t(bn, fuse_w_ref[...]) + fuse_b_ref[...][None, :])
    xp_ref[...] = jnp.pad(x, ((0, 0), (0, XG - 40)))


def _tc1(gene, miRNA, rna_w, rna_b, mi_w, mi_b,
         rnl_w, rnl_b, rl_w, rl_b, rg_w, rg_b,
         mnl_w, mnl_b, ml_w, ml_b, mg_w, mg_b,
         rna_lin_w, mi_lin_w, bn_g, bn_b, fuse_w, fuse_b):
    def full(a):
        return pl.BlockSpec(a.shape, lambda i: (0,) * a.ndim)
    in_specs = [
        pl.BlockSpec((RB1, 6000), lambda i: (i, 0)),
        pl.BlockSpec((RB1, 600), lambda i: (i, 0)),
        full(rna_w), full(rna_b), full(mi_w), full(mi_b),
        full(rnl_w), full(rnl_b), full(rl_w), full(rl_b), full(rg_w), full(rg_b),
        full(mnl_w), full(mnl_b), full(ml_w), full(ml_b), full(mg_w), full(mg_b),
        full(rna_lin_w), full(mi_lin_w), full(bn_g), full(bn_b),
        full(fuse_w), full(fuse_b),
    ]
    out_specs = [
        pl.BlockSpec((RB1, 40), lambda i: (i, 0)),
        pl.BlockSpec((RB1, 40), lambda i: (i, 0)),
        pl.BlockSpec((RB1, 60), lambda i: (i, 0)),
        pl.BlockSpec((RB1, XG), lambda i: (i, 0)),
    ]
    out_shape = [
        jax.ShapeDtypeStruct((B, 40), jnp.float32),
        jax.ShapeDtypeStruct((B, 40), jnp.float32),
        jax.ShapeDtypeStruct((B, 60), jnp.float32),
        jax.ShapeDtypeStruct((B, XG), jnp.float32),
    ]
    return pl.pallas_call(
        _tc1_body, grid=(NB1,), in_specs=in_specs, out_specs=out_specs,
        out_shape=out_shape)(
            gene, miRNA, rna_w, rna_b, mi_w, mi_b,
            rnl_w, rnl_b, rl_w, rl_b, rg_w, rg_b,
            mnl_w, mnl_b, ml_w, ml_b, mg_w, mg_b,
            rna_lin_w, mi_lin_w, bn_g, bn_b, fuse_w, fuse_b)


def _tc2_body(rnb_ref, rn_ref, mnb_ref, mn_ref, fnb_ref, fn_ref,
              idxt_ref, at_ref, wt_ref, u0_ref, u1_ref, u2_ref):
    u_refs = (u0_ref, u1_ref, u2_ref)
    pid = pl.program_id(0)
    iota = lax.broadcasted_iota(jnp.int32, (RB, B), 1)
    for g, (blk_ref, ful_ref) in enumerate(
            [(rnb_ref, rn_ref), (mnb_ref, mn_ref), (fnb_ref, fn_ref)]):
        fb = blk_ref[...]
        ff = ful_ref[...]
        sqf = jnp.sum(ff * ff, axis=1)
        sqb = jnp.sum(fb * fb, axis=1)
        d2 = (sqb[:, None] + sqf[None, :]) - 2.0 * _dot_t(fb, ff)
        k = KS[g]
        for t in range(k):
            m = jnp.min(d2, axis=1)
            eq = d2 == m[:, None]
            idxv = jnp.min(jnp.where(eq, iota, B), axis=1)
            d2 = jnp.where(eq, jnp.inf, d2)
            idxt_ref[S0[g] + t, :] = idxv
            at_ref[S0[g] + t, :] = m
            wt_ref[S0[g] + t, :] = jnp.exp(-m / 20.0)
            if t == k - 1:
                u_refs[g][...] = m
    idxt_ref[NSLOT, :] = pid * RB + lax.iota(jnp.int32, RB)


def _tc2(rn, mn, fn):
    def blk(d):
        return pl.BlockSpec((RB, d), lambda i: (i, 0))

    def ful(d):
        return pl.BlockSpec((B, d), lambda i: (0, 0))
    return pl.pallas_call(
        _tc2_body, grid=(NB,),
        in_specs=[blk(40), ful(40), blk(40), ful(40), blk(60), ful(60)],
        out_specs=[
            pl.BlockSpec((NSLOTP, RB), lambda i: (0, i)),
            pl.BlockSpec((NSLOT, RB), lambda i: (0, i)),
            pl.BlockSpec((NSLOT, RB), lambda i: (0, i)),
            pl.BlockSpec((RB,), lambda i: (i,)),
            pl.BlockSpec((RB,), lambda i: (i,)),
            pl.BlockSpec((RB,), lambda i: (i,)),
        ],
        out_shape=[
            jax.ShapeDtypeStruct((NSLOTP, B), jnp.int32),
            jax.ShapeDtypeStruct((NSLOT, B), jnp.float32),
            jax.ShapeDtypeStruct((NSLOT, B), jnp.float32),
            jax.ShapeDtypeStruct((B,), jnp.float32),
            jax.ShapeDtypeStruct((B,), jnp.float32),
            jax.ShapeDtypeStruct((B,), jnp.float32),
        ])(rn, rn, mn, mn, fn, fn)


_SC_MESH = plsc.VectorSubcoreMesh(core_axis_name="c", subcore_axis_name="s")


def _wid_base():
    return (lax.axis_index("s") * NC + lax.axis_index("c")) * RPW


def _sc1_body(idxt_hbm, at_hbm, wt_hbm, u0_hbm, u1_hbm, u2_hbm,
              mvt_hbm, c0_hbm, c1_hbm, c2_hbm,
              u_v, idx_v, a_v, w_v, mv_v, c_v):
    base = _wid_base()
    for g, u_hbm in enumerate((u0_hbm, u1_hbm, u2_hbm)):
        pltpu.sync_copy(u_hbm, u_v.at[pl.ds(g * B, B)])
    pltpu.sync_copy(idxt_hbm.at[:, pl.ds(base, RPW)], idx_v)
    pltpu.sync_copy(at_hbm.at[:, pl.ds(base, RPW)], a_v)
    pltpu.sync_copy(wt_hbm.at[:, pl.ds(base, RPW)], w_v)
    for g in range(3):
        goff = jnp.full((LN,), g * B, jnp.int32)
        for ch in range(NCH):
            sl = pl.ds(ch * LN, LN)
            csum = jnp.zeros((LN,), jnp.float32)
            for s in range(S0[g], S0[g] + KS[g]):
                j = idx_v[s, sl]
                uj = plsc.load_gather(u_v, [j + goff])
                keep = a_v[s, sl] <= uj
                mv = jnp.where(keep, w_v[s, sl], 0.0)
                mv_v[s, sl] = mv
                csum = csum + mv
            c_v[pl.ds(g * RPW + ch * LN, LN)] = csum
    pltpu.sync_copy(mv_v, mvt_hbm.at[:, pl.ds(base, RPW)])
    for g, c_hbm in enumerate((c0_hbm, c1_hbm, c2_hbm)):
        pltpu.sync_copy(c_v.at[pl.ds(g * RPW, RPW)], c_hbm.at[pl.ds(base, RPW)])


def _sc1(idxt, at, wt, u0, u1, u2):
    f = functools.partial(
        pl.kernel, mesh=_SC_MESH,
        compiler_params=pltpu.CompilerParams(needs_layout_passes=False),
        out_type=[
            jax.ShapeDtypeStruct((NSLOT, B), jnp.float32),
            jax.ShapeDtypeStruct((B,), jnp.float32),
            jax.ShapeDtypeStruct((B,), jnp.float32),
            jax.ShapeDtypeStruct((B,), jnp.float32),
        ],
        scratch_types=[
            pltpu.VMEM((3 * B,), jnp.float32),
            pltpu.VMEM((NSLOTP, RPW), jnp.int32),
            pltpu.VMEM((NSLOT, RPW), jnp.float32),
            pltpu.VMEM((NSLOT, RPW), jnp.float32),
            pltpu.VMEM((NSLOT, RPW), jnp.float32),
            pltpu.VMEM((3 * RPW,), jnp.float32),
        ])
    return f(_sc1_body)(idxt, at, wt, u0, u1, u2)


def _sc2_body(idxt_hbm, mvt_hbm, c0_hbm, c1_hbm, c2_hbm,
              wnt_hbm, ds2_hbm,
              c_v, idx_v, mv_v, wn_v, ds2_v):
    base = _wid_base()
    for g, c_hbm in enumerate((c0_hbm, c1_hbm, c2_hbm)):
        pltpu.sync_copy(c_hbm, c_v.at[pl.ds(g * B, B)])
    pltpu.sync_copy(idxt_hbm.at[:, pl.ds(base, RPW)], idx_v)
    pltpu.sync_copy(mvt_hbm.at[:, pl.ds(base, RPW)], mv_v)
    for ch in range(NCH):
        sl = pl.ds(ch * LN, LN)
        dsum = jnp.full((LN,), 1.0, jnp.float32)
        for g in range(3):
            goff = jnp.full((LN,), g * B, jnp.int32)
            for s in range(S0[g], S0[g] + KS[g]):
                j = idx_v[s, sl]
                cj = plsc.load_gather(c_v, [j + goff])
                wn = mv_v[s, sl] / cj
                wn_v[s, sl] = wn
                dsum = dsum + wn
        wn_v[NSLOT, sl] = jnp.full((LN,), 1.0, jnp.float32)
        ds2_v[sl] = 1.0 / (dsum + EPS)
    pltpu.sync_copy(wn_v, wnt_hbm.at[:, pl.ds(base, RPW)])
    pltpu.sync_copy(ds2_v, ds2_hbm.at[pl.ds(base, RPW)])


def _sc2(idxt, mvt, c0, c1, c2):
    f = functools.partial(
        pl.kernel, mesh=_SC_MESH,
        compiler_params=pltpu.CompilerParams(needs_layout_passes=False),
        out_type=[
            jax.ShapeDtypeStruct((NSLOTP, B), jnp.float32),
            jax.ShapeDtypeStruct((B,), jnp.float32),
        ],
        scratch_types=[
            pltpu.VMEM((3 * B,), jnp.float32),
            pltpu.VMEM((NSLOTP, RPW), jnp.int32),
            pltpu.VMEM((NSLOT, RPW), jnp.float32),
            pltpu.VMEM((NSLOTP, RPW), jnp.float32),
            pltpu.VMEM((RPW,), jnp.float32),
        ])
    return f(_sc2_body)(idxt, mvt, c0, c1, c2)


def _sc3_body(idxt_hbm, wnt_hbm, ds2_hbm, xp_hbm,
              h_hbm,
              ds2_v, idx_v, wn_v, xg_v, h_v):
    base = _wid_base()
    pltpu.sync_copy(ds2_hbm, ds2_v)
    pltpu.sync_copy(idxt_hbm.at[:, pl.ds(base, RPW)], idx_v)
    pltpu.sync_copy(wnt_hbm.at[:, pl.ds(base, RPW)], wn_v)
    for r in range(RPW):
        for q in range(XP // LN):
            h_v[r, pl.ds(q * LN, LN)] = jnp.zeros((LN,), jnp.float32)

    @pl.loop(0, NSLOTP)
    def _slot(s):
        pltpu.sync_copy(xp_hbm.at[idx_v.at[s]], xg_v)
        for ch in range(NCH):
            sl = pl.ds(ch * LN, LN)
            j = idx_v[s, sl]
            dj = plsc.load_gather(ds2_v, [j])
            coef = wn_v[s, sl] * dj
            for t in range(LN):
                cf = coef[t]
                r = ch * LN + t
                for q in range(XP // LN):
                    qs = pl.ds(q * LN, LN)
                    h_v[r, qs] = h_v[r, qs] + cf * xg_v[r, qs]
    pltpu.sync_copy(h_v, h_hbm.at[pl.ds(base, RPW), :])


def _sc3(idxt, wnt, ds2, xp):
    f = functools.partial(
        pl.kernel, mesh=_SC_MESH,
        compiler_params=pltpu.CompilerParams(needs_layout_passes=False),
        out_type=jax.ShapeDtypeStruct((B, XP), jnp.float32),
        scratch_types=[
            pltpu.VMEM((B,), jnp.float32),
            pltpu.VMEM((NSLOTP, RPW), jnp.int32),
            pltpu.VMEM((NSLOTP, RPW), jnp.float32),
            pltpu.VMEM((RPW, XG), jnp.float32),
            pltpu.VMEM((RPW, XP), jnp.float32),
        ])
    return f(_sc3_body)(idxt, wnt, ds2, xp)


def _tc3_body(h_ref, c1_w_ref, c1_b_ref, c2_w_ref, c2_b_ref, out_ref):
    h = h_ref[...][:, :40]
    a1 = jax.nn.relu(_dot_t(h, c1_w_ref[...]) + c1_b_ref[...][None, :])
    res = jnp.sum(a1 * c2_w_ref[...], axis=1, keepdims=True)
    out_ref[...] = res + c2_b_ref[...]


def _tc3(ht, c1_w, c1_b, c2_w, c2_b):
    def full(a):
        return pl.BlockSpec(a.shape, lambda: (0,) * a.ndim)
    return pl.pallas_call(
        _tc3_body,
        in_specs=[full(ht), full(c1_w), full(c1_b), full(c2_w), full(c2_b)],
        out_specs=pl.BlockSpec((B, 1), lambda: (0, 0)),
        out_shape=jax.ShapeDtypeStruct((B, 1), jnp.float32),
    )(ht, c1_w, c1_b, c2_w, c2_b)


def kernel(gene, miRNA, rna_w, rna_b, mi_w, mi_b,
           hw_r_nl_w, hw_r_nl_b, hw_r_l_w, hw_r_l_b, hw_r_g_w, hw_r_g_b,
           hw_m_nl_w, hw_m_nl_b, hw_m_l_w, hw_m_l_b, hw_m_g_w, hw_m_g_b,
           rna_lin_w, mi_lin_w, bn_g, bn_b, fuse_w, fuse_b,
           c1_w, c1_b, c2_w, c2_b):
    gene = gene.reshape(gene.shape[0], -1)
    miRNA = miRNA.reshape(miRNA.shape[0], -1)
    rn, mn, fn, xp = _tc1(gene, miRNA, rna_w, rna_b, mi_w, mi_b,
                          hw_r_nl_w, hw_r_nl_b, hw_r_l_w, hw_r_l_b,
                          hw_r_g_w, hw_r_g_b,
                          hw_m_nl_w, hw_m_nl_b, hw_m_l_w, hw_m_l_b,
                          hw_m_g_w, hw_m_g_b,
                          rna_lin_w, mi_lin_w, bn_g, bn_b, fuse_w, fuse_b)
    idxt, at, wt, u0, u1, u2 = _tc2(rn, mn, fn)
    mvt, c0, c1, c2 = _sc1(idxt, at, wt, u0, u1, u2)
    wnt, ds2 = _sc2(idxt, mvt, c0, c1, c2)
    h = _sc3(idxt, wnt, ds2, xp)
    return _tc3(h, c1_w, c1_b, c2_w, c2_b)
